# baseline (speedup 1.0000x reference)
import functools

import jax
import jax.numpy as jnp
from jax import lax
from jax.experimental import pallas as pl
from jax.experimental.pallas import tpu as pltpu
from jax.experimental.pallas import tpu_sc as plsc

B = 4096
V = 100000
D = 32
T = 26
NUM = 13

TT = 13
NC, NS = 2, 16
NW = NC * NS
BPW = B // NW
LANES = 16


def _detile_body(tab_ref, out_ref):
    for k in range(D):
        out_ref[pl.ds(k * V, V)] = tab_ref[0, k, :]


def _detile(tab_t):
    return pl.pallas_call(
        _detile_body,
        grid=(TT,),
        in_specs=[pl.BlockSpec((1, D, V), lambda t: (t, 0, 0))],
        out_specs=pl.BlockSpec((D * V,), lambda t: (t,)),
        out_shape=jax.ShapeDtypeStruct((TT * D * V,), jnp.float32),
        compiler_params=pltpu.CompilerParams(
            vmem_limit_bytes=120 * 1024 * 1024),
    )(tab_t)


@functools.lru_cache(maxsize=None)
def _make_sc_gather():
    mesh = plsc.VectorSubcoreMesh(
        core_axis_name="c", subcore_axis_name="s",
        num_cores=NC, num_subcores=NS)

    @functools.partial(
        pl.kernel,
        out_type=jax.ShapeDtypeStruct((NW, TT, D, BPW), jnp.float32),
        mesh=mesh,
        scratch_types=[
            pltpu.VMEM((TT, BPW), jnp.int32),
            pltpu.VMEM((2, D, BPW), jnp.int32),
            pltpu.VMEM((2, D, BPW), jnp.float32),
            pltpu.SemaphoreType.DMA,
            pltpu.SemaphoreType.DMA,
            pltpu.SemaphoreType.DMA,
            pltpu.SemaphoreType.DMA,
        ],
        compiler_params=pltpu.CompilerParams(use_tc_tiling_on_sc=False),
    )
    def _sc_gather(flat_hbm, gidx_hbm, out_hbm, idx_v, fidx_v, rows_v,
                   g0, g1, w0, w1):
        wid = lax.axis_index("s") * NC + lax.axis_index("c")
        pltpu.sync_copy(gidx_hbm.at[wid], idx_v)
        gsem = [g0, g1]
        wsem = [w0, w1]
        pend_g = [None, None]
        pend_w = [None, None]
        for t in range(TT):
            q = t & 1
            if pend_w[q] is not None:
                pend_w[q].wait()
            for d in range(D):
                base = (t * D + d) * V
                for c in range(BPW // LANES):
                    fidx_v[q, d, pl.ds(c * LANES, LANES)] = (
                        idx_v[t, pl.ds(c * LANES, LANES)] + base)
            pend_g[q] = [
                pltpu.async_copy(
                    flat_hbm.at[fidx_v.at[q, d]], rows_v.at[q, d], gsem[q])
                for d in range(D)
            ]
            p = 1 - q
            if pend_g[p] is not None:
                for cp in pend_g[p]:
                    cp.wait()
                pend_g[p] = None
                pend_w[p] = pltpu.async_copy(
                    rows_v.at[p], out_hbm.at[wid, t - 1], wsem[p])
        q = (TT - 1) & 1
        for cp in pend_g[q]:
            cp.wait()
        pltpu.async_copy(rows_v.at[q], out_hbm.at[wid, TT - 1], wsem[q]).wait()
        if pend_w[1 - q] is not None:
            pend_w[1 - q].wait()

    return _sc_gather


def _mlp_body(emba_ref, embb_ref, num_ref, w1a_ref, w1b_ref, w1n_ref,
              b1_ref, w2_ref, b2_ref, w3_ref, b3_ref, w4_ref, b4_ref,
              out_ref):
    kta = emba_ref[0].reshape(TT * D, BPW)
    ktb = embb_ref[0].reshape(TT * D, BPW)
    x1 = lax.dot_general(kta, w1a_ref[...], (((0,), (0,)), ((), ())),
                         preferred_element_type=jnp.float32)
    x1 = x1 + lax.dot_general(ktb, w1b_ref[...], (((0,), (0,)), ((), ())),
                              preferred_element_type=jnp.float32)
    x1 = x1 + jnp.dot(num_ref[...], w1n_ref[...],
                      preferred_element_type=jnp.float32)
    h = jnp.maximum(x1 + b1_ref[...], 0.0)
    h = jnp.maximum(
        jnp.dot(h, w2_ref[...], preferred_element_type=jnp.float32)
        + b2_ref[...], 0.0)
    h = jnp.maximum(
        jnp.dot(h, w3_ref[...], preferred_element_type=jnp.float32)
        + b3_ref[...], 0.0)
    out_ref[...] = (
        jnp.dot(h, w4_ref[...], preferred_element_type=jnp.float32)
        + b4_ref[...])


def _mlp(emba, embb, num, w1a, w1b, w1n, b1, w2, b2, w3, b3, w4, b4,
         *, interpret=False):
    full = lambda shape: pl.BlockSpec(shape, lambda i: (0,) * len(shape))
    return pl.pallas_call(
        _mlp_body,
        grid=(NW,),
        in_specs=[
            pl.BlockSpec((1, TT, D, BPW), lambda i: (i, 0, 0, 0)),
            pl.BlockSpec((1, TT, D, BPW), lambda i: (i, 0, 0, 0)),
            pl.BlockSpec((BPW, NUM), lambda i: (i, 0)),
            full((TT * D, 512)),
            full((TT * D, 512)),
            full((NUM, 512)),
            full((1, 512)),
            full((512, 256)),
            full((1, 256)),
            full((256, 128)),
            full((1, 128)),
            full((128, 1)),
            full((1, 1)),
        ],
        out_specs=pl.BlockSpec((BPW, 1), lambda i: (i, 0)),
        out_shape=jax.ShapeDtypeStruct((B, 1), jnp.float32),
        interpret=interpret,
    )(emba, embb, num, w1a, w1b, w1n, b1, w2, b2, w3, b3, w4, b4)


def kernel(numerical_features, cat_features, tables, W1, b1, W2, b2, W3, b3,
           W4, b4):
    cat = cat_features.reshape(T, B).astype(jnp.int32)
    sc_gather = _make_sc_gather()

    embs = []
    for g in range(2):
        tab = tables[g * TT:(g + 1) * TT]
        flat = _detile(tab.transpose(0, 2, 1))
        gidx = (cat[g * TT:(g + 1) * TT]
                .reshape(TT, NW, BPW).transpose(1, 0, 2))
        embs.append(sc_gather(flat, gidx))

    w1a = W1[: TT * D]
    w1b = W1[TT * D: T * D]
    w1n = W1[T * D:]
    return _mlp(embs[0], embs[1], numerical_features, w1a, w1b, w1n,
                b1.reshape(1, 512), W2, b2.reshape(1, 256),
                W3, b3.reshape(1, 128), W4, b4.reshape(1, 1))


# speedup vs baseline: 1.4825x; 1.4825x over previous
import functools

import jax
import jax.numpy as jnp
from jax import lax
from jax.experimental import pallas as pl
from jax.experimental.pallas import tpu as pltpu
from jax.experimental.pallas import tpu_sc as plsc

B = 4096
V = 100000
D = 32
T = 26
NUM = 13

NC, NS = 2, 16
NW = NC * NS
BPW = B // NW
LANES = 16


def _detile_body(tab_ref, out_ref):
    for k in range(D):
        out_ref[pl.ds(k * V, V)] = tab_ref[0, k, :]


def _detile(tab_t):
    return pl.pallas_call(
        _detile_body,
        grid=(T,),
        in_specs=[pl.BlockSpec((1, D, V), lambda t: (t, 0, 0))],
        out_specs=pl.BlockSpec((D * V,), lambda t: (t,)),
        out_shape=jax.ShapeDtypeStruct((T * D * V,), jnp.float32),
        compiler_params=pltpu.CompilerParams(
            vmem_limit_bytes=120 * 1024 * 1024),
    )(tab_t)


@functools.lru_cache(maxsize=None)
def _make_sc_gather():
    mesh = plsc.VectorSubcoreMesh(
        core_axis_name="c", subcore_axis_name="s",
        num_cores=NC, num_subcores=NS)

    @functools.partial(
        pl.kernel,
        out_type=jax.ShapeDtypeStruct((NW, T, D, BPW), jnp.float32),
        mesh=mesh,
        scratch_types=[
            pltpu.VMEM((T, BPW), jnp.int32),
            pltpu.VMEM((2, D, BPW), jnp.int32),
            pltpu.VMEM((2, D, BPW), jnp.float32),
            pltpu.SemaphoreType.DMA,
            pltpu.SemaphoreType.DMA,
            pltpu.SemaphoreType.DMA,
            pltpu.SemaphoreType.DMA,
        ],
        compiler_params=pltpu.CompilerParams(use_tc_tiling_on_sc=False),
    )
    def _sc_gather(flat_hbm, gidx_hbm, out_hbm, idx_v, fidx_v, rows_v,
                   g0, g1, w0, w1):
        wid = lax.axis_index("s") * NC + lax.axis_index("c")
        pltpu.sync_copy(gidx_hbm.at[wid], idx_v)
        gsem = [g0, g1]
        wsem = [w0, w1]

        def fill_and_fire(t, q):
            for d in range(D):
                base_c = d * V
                for c in range(BPW // LANES):
                    fidx_v[q, d, pl.ds(c * LANES, LANES)] = (
                        idx_v[t, pl.ds(c * LANES, LANES)] + (t * (D * V)
                                                             + base_c))
            return [
                pltpu.async_copy(
                    flat_hbm.at[fidx_v.at[q, d]], rows_v.at[q, d], gsem[q])
                for d in range(D)
            ]

        def pair(i, _):
            t0 = i * 2
            t1 = t0 + 1

            @pl.when(i > 0)
            def _():
                for q in range(2):
                    pltpu.make_async_copy(
                        rows_v.at[q], out_hbm.at[wid, 0], wsem[q]).wait()

            cps0 = fill_and_fire(t0, 0)
            cps1 = fill_and_fire(t1, 1)
            for cp in cps0:
                cp.wait()
            pltpu.async_copy(rows_v.at[0], out_hbm.at[wid, t0], wsem[0])
            for cp in cps1:
                cp.wait()
            pltpu.async_copy(rows_v.at[1], out_hbm.at[wid, t1], wsem[1])
            return ()

        lax.fori_loop(0, T // 2, pair, (), unroll=False)
        for q in range(2):
            pltpu.make_async_copy(
                rows_v.at[q], out_hbm.at[wid, 0], wsem[q]).wait()

    return _sc_gather


def _mlp_body(emb_ref, num_ref, w1e_ref, w1n_ref, b1_ref, w2_ref, b2_ref,
              w3_ref, b3_ref, w4_ref, b4_ref, out_ref):
    kt = emb_ref[0].reshape(T * D, BPW)
    x1 = lax.dot_general(kt, w1e_ref[...], (((0,), (0,)), ((), ())),
                         preferred_element_type=jnp.float32)
    x1 = x1 + jnp.dot(num_ref[...], w1n_ref[...],
                      preferred_element_type=jnp.float32)
    h = jnp.maximum(x1 + b1_ref[...], 0.0)
    h = jnp.maximum(
        jnp.dot(h, w2_ref[...], preferred_element_type=jnp.float32)
        + b2_ref[...], 0.0)
    h = jnp.maximum(
        jnp.dot(h, w3_ref[...], preferred_element_type=jnp.float32)
        + b3_ref[...], 0.0)
    out_ref[...] = (
        jnp.dot(h, w4_ref[...], preferred_element_type=jnp.float32)
        + b4_ref[...])


def _mlp(emb5, num, w1e, w1n, b1, w2, b2, w3, b3, w4, b4, *, interpret=False):
    full = lambda shape: pl.BlockSpec(shape, lambda i: (0,) * len(shape))
    return pl.pallas_call(
        _mlp_body,
        grid=(NW,),
        in_specs=[
            pl.BlockSpec((1, T, D, BPW), lambda i: (i, 0, 0, 0)),
            pl.BlockSpec((BPW, NUM), lambda i: (i, 0)),
            full((T * D, 512)),
            full((NUM, 512)),
            full((1, 512)),
            full((512, 256)),
            full((1, 256)),
            full((256, 128)),
            full((1, 128)),
            full((128, 1)),
            full((1, 1)),
        ],
        out_specs=pl.BlockSpec((BPW, 1), lambda i: (i, 0)),
        out_shape=jax.ShapeDtypeStruct((B, 1), jnp.float32),
        interpret=interpret,
    )(emb5, num, w1e, w1n, b1, w2, b2, w3, b3, w4, b4)


def kernel(numerical_features, cat_features, tables, W1, b1, W2, b2, W3, b3,
           W4, b4):
    flat = _detile(tables.transpose(0, 2, 1))

    cat = cat_features.reshape(T, B).astype(jnp.int32)
    gidx = cat.reshape(T, NW, BPW).transpose(1, 0, 2)

    emb5 = _make_sc_gather()(flat, gidx)

    w1e = W1[: T * D]
    w1n = W1[T * D:]
    return _mlp(emb5, numerical_features, w1e, w1n, b1.reshape(1, 512),
                W2, b2.reshape(1, 256), W3, b3.reshape(1, 128),
                W4, b4.reshape(1, 1))
